# Initial kernel scaffold; baseline (speedup 1.0000x reference)
import functools

import jax
import jax.numpy as jnp
from jax import lax
from jax.experimental import pallas as pl
from jax.experimental.pallas import tpu as pltpu, tpu_sc as plsc

D = 32
SEQ = 50
NUM_WORKERS = 32
IDX_COLS = 80
UNIT_BAGS = 8
UNIT_ROWS = UNIT_BAGS * SEQ
GATHERS_PER_UNIT = UNIT_ROWS // IDX_COLS
NBUF = 2


def _bag_kernel(batch, ids_hbm, table_hbm, out_hbm,
                idx_v, rows0, rows1, out_v, sem0, sem1):
    bags_per_w = batch // NUM_WORKERS
    num_units = bags_per_w // UNIT_BAGS
    idx_rows_per_w = bags_per_w * SEQ // IDX_COLS

    wid = lax.axis_index("s") * 2 + lax.axis_index("c")
    idx_row0 = wid * idx_rows_per_w
    bag0 = wid * bags_per_w

    pltpu.sync_copy(ids_hbm.at[pl.ds(idx_row0, idx_rows_per_w)], idx_v)

    rows_bufs = (rows0, rows1)
    sems = (sem0, sem1)

    def fire(u, rows_ref, sem):
        for g in range(GATHERS_PER_UNIT):
            pltpu.make_async_copy(
                table_hbm.at[idx_v.at[u * GATHERS_PER_UNIT + g]],
                rows_ref.at[pl.ds(g * IDX_COLS, IDX_COLS)],
                sem,
            ).start()

    def drain(rows_ref, sem):
        pltpu.make_async_copy(
            table_hbm.at[pl.ds(0, UNIT_ROWS)], rows_ref, sem
        ).wait()

    inv = jnp.float32(1.0 / SEQ)
    lo = pl.ds(0, 16)
    hi = pl.ds(16, 16)

    def reduce_unit(rows_ref, u):
        def bag_body(b, carry):
            base = b * SEQ
            a0 = rows_ref[base, lo]
            a1 = rows_ref[base, hi]
            b0 = rows_ref[base + 1, lo]
            b1 = rows_ref[base + 1, hi]
            for s in range(2, SEQ, 2):
                a0 = a0 + rows_ref[base + s, lo]
                a1 = a1 + rows_ref[base + s, hi]
                b0 = b0 + rows_ref[base + s + 1, lo]
                b1 = b1 + rows_ref[base + s + 1, hi]
            bag = u * UNIT_BAGS + b
            out_v[bag, lo] = (a0 + b0) * inv
            out_v[bag, hi] = (a1 + b1) * inv
            return carry
        lax.fori_loop(0, UNIT_BAGS, bag_body, 0)

    for p in range(NBUF):
        fire(p, rows_bufs[p], sems[p])

    def outer(k, carry):
        for p in range(NBUF):
            u = k * NBUF + p
            drain(rows_bufs[p], sems[p])
            reduce_unit(rows_bufs[p], u)

            @pl.when(u + NBUF < num_units)
            def _():
                fire(u + NBUF, rows_bufs[p], sems[p])
        return carry

    lax.fori_loop(0, num_units // NBUF, outer, 0)

    pltpu.sync_copy(out_v, out_hbm.at[pl.ds(bag0, bags_per_w)])


@jax.jit
def _bag_of_words(ids2d, table):
    batch = ids2d.shape[0] * ids2d.shape[1] // SEQ
    bags_per_w = batch // NUM_WORKERS
    idx_rows_per_w = bags_per_w * SEQ // IDX_COLS
    grid_kernel = functools.partial(
        pl.kernel,
        mesh=plsc.VectorSubcoreMesh(core_axis_name="c", subcore_axis_name="s"),
        out_type=jax.ShapeDtypeStruct((batch, D), jnp.float32),
        scratch_types=[
            pltpu.VMEM((idx_rows_per_w, IDX_COLS), jnp.int32),
            pltpu.VMEM((UNIT_ROWS, D), jnp.float32),
            pltpu.VMEM((UNIT_ROWS, D), jnp.float32),
            pltpu.VMEM((bags_per_w, D), jnp.float32),
            pltpu.SemaphoreType.DMA,
            pltpu.SemaphoreType.DMA,
        ],
    )
    return grid_kernel(functools.partial(_bag_kernel, batch))(ids2d, table)


def kernel(token_ids, table):
    batch, seq = token_ids.shape
    assert seq == SEQ and table.shape[1] == D
    ids2d = token_ids.astype(jnp.int32).reshape(batch * seq // IDX_COLS, IDX_COLS)
    return _bag_of_words(ids2d, table)


# speedup vs baseline: 2.8465x; 2.8465x over previous
import functools

import jax
import jax.numpy as jnp
from jax import lax
from jax.experimental import pallas as pl
from jax.experimental.pallas import tpu as pltpu, tpu_sc as plsc

D = 32
SEQ = 50
NUM_WORKERS = 32
IDX_COLS = 80
UNIT_BAGS = 8
UNIT_ROWS = UNIT_BAGS * SEQ
GATHERS_PER_UNIT = UNIT_ROWS // IDX_COLS
NBUF = 2


def _bag_kernel(batch, ids_hbm, table_hbm, out_hbm,
                idx_v, rows0, rows1, out_v, sem0, sem1):
    bags_per_w = batch // NUM_WORKERS
    num_units = bags_per_w // UNIT_BAGS
    idx_rows_per_w = bags_per_w * SEQ // IDX_COLS

    wid = lax.axis_index("s") * 2 + lax.axis_index("c")
    idx_row0 = wid * idx_rows_per_w
    bag0 = wid * bags_per_w

    pltpu.sync_copy(ids_hbm.at[pl.ds(idx_row0, idx_rows_per_w)], idx_v)

    rows_bufs = (rows0, rows1)
    sems = (sem0, sem1)

    def fire(u, rows_ref, sem):
        for g in range(GATHERS_PER_UNIT):
            pltpu.make_async_copy(
                table_hbm.at[idx_v.at[u * GATHERS_PER_UNIT + g]],
                rows_ref.at[pl.ds(g * IDX_COLS, IDX_COLS)],
                sem,
            ).start()

    def drain(rows_ref, sem):
        pltpu.make_async_copy(
            table_hbm.at[pl.ds(0, UNIT_ROWS)], rows_ref, sem
        ).wait()

    inv = jnp.float32(1.0 / SEQ)
    lo = pl.ds(0, 16)
    hi = pl.ds(16, 16)

    def reduce_unit(rows_ref, u):
        def bag_body(b, carry):
            base = b * SEQ
            a0 = rows_ref[base, lo]
            a1 = rows_ref[base, hi]
            b0 = rows_ref[base + 1, lo]
            b1 = rows_ref[base + 1, hi]
            for s in range(2, SEQ, 2):
                a0 = a0 + rows_ref[base + s, lo]
                a1 = a1 + rows_ref[base + s, hi]
                b0 = b0 + rows_ref[base + s + 1, lo]
                b1 = b1 + rows_ref[base + s + 1, hi]
            bag = u * UNIT_BAGS + b
            out_v[bag, lo] = (a0 + b0) * inv
            out_v[bag, hi] = (a1 + b1) * inv
            return carry
        lax.fori_loop(0, UNIT_BAGS, bag_body, 0)

    for p in range(NBUF):
        fire(p, rows_bufs[p], sems[p])

    def outer(k, carry):
        for p in range(NBUF):
            u = k * NBUF + p
            drain(rows_bufs[p], sems[p])
            reduce_unit(rows_bufs[p], u)

            @pl.when(u + NBUF < num_units)
            def _():
                fire(u + NBUF, rows_bufs[p], sems[p])
        return carry

    lax.fori_loop(0, num_units // NBUF, outer, 0)

    pltpu.sync_copy(out_v, out_hbm.at[pl.ds(bag0, bags_per_w)])


@jax.jit
def _bag_of_words(ids2d, table):
    batch = ids2d.shape[0] * ids2d.shape[1] // SEQ
    bags_per_w = batch // NUM_WORKERS
    idx_rows_per_w = bags_per_w * SEQ // IDX_COLS
    grid_kernel = functools.partial(
        pl.kernel,
        mesh=plsc.VectorSubcoreMesh(core_axis_name="c", subcore_axis_name="s"),
        out_type=jax.ShapeDtypeStruct((batch, D), jnp.float32),
        scratch_types=[
            pltpu.VMEM((idx_rows_per_w, IDX_COLS), jnp.int32),
            pltpu.VMEM((UNIT_ROWS, D), jnp.float32),
            pltpu.VMEM((UNIT_ROWS, D), jnp.float32),
            pltpu.VMEM((bags_per_w, D), jnp.float32),
            pltpu.SemaphoreType.DMA,
            pltpu.SemaphoreType.DMA,
        ],
        compiler_params=pltpu.CompilerParams(use_tc_tiling_on_sc=False),
    )
    return grid_kernel(functools.partial(_bag_kernel, batch))(ids2d, table)


def kernel(token_ids, table):
    batch, seq = token_ids.shape
    assert seq == SEQ and table.shape[1] == D
    ids2d = token_ids.astype(jnp.int32).reshape(batch * seq // IDX_COLS, IDX_COLS)
    return _bag_of_words(ids2d, table)


# speedup vs baseline: 3.0167x; 1.0598x over previous
import functools

import jax
import jax.numpy as jnp
from jax import lax
from jax.experimental import pallas as pl
from jax.experimental.pallas import tpu as pltpu, tpu_sc as plsc

D = 32
SEQ = 50
NUM_WORKERS = 32
UNIT_BAGS = 128
NBUF = 2


def _bag_kernel(batch, ids_hbm, table_hbm, out_hbm,
                idx_v, acc0, acc1, out_v, sem0, sem1):
    bags_per_w = batch // NUM_WORKERS
    units_per_w = bags_per_w // UNIT_BAGS

    wid = lax.axis_index("s") * 2 + lax.axis_index("c")
    unit0 = wid * units_per_w
    bag0 = wid * bags_per_w

    pltpu.sync_copy(ids_hbm.at[:, pl.ds(unit0, units_per_w), :], idx_v)

    accs = (acc0, acc1)
    sems = (sem0, sem1)

    inv = jnp.float32(1.0 / SEQ)
    lo = pl.ds(0, 16)
    hi = pl.ds(16, 16)
    zvec = jnp.zeros((16,), jnp.float32)

    def zero(acc):
        def body(r, c):
            acc[r, lo] = zvec
            acc[r, hi] = zvec
            return c
        lax.fori_loop(0, UNIT_BAGS, body, 0)

    def fire(u, acc, sem):
        def step(st, c):
            for j in range(10):
                pltpu.async_copy(
                    table_hbm.at[idx_v.at[st * 10 + j, u]], acc, sem, add=True)
            return c
        lax.fori_loop(0, SEQ // 10, step, 0)

    def drain(acc, sem):
        def body(i, c):
            pltpu.make_async_copy(
                table_hbm.at[pl.ds(0, UNIT_BAGS)], acc, sem).wait()
            return c
        lax.fori_loop(0, SEQ, body, 0)

    def scale_out(u, acc):
        def body(r, c):
            bag = u * UNIT_BAGS + r
            out_v[bag, lo] = acc[r, lo] * inv
            out_v[bag, hi] = acc[r, hi] * inv
            return c
        lax.fori_loop(0, UNIT_BAGS, body, 0)

    for p in range(NBUF):
        zero(accs[p])
        fire(p, accs[p], sems[p])

    for u in range(units_per_w):
        p = u % NBUF
        drain(accs[p], sems[p])
        scale_out(u, accs[p])
        if u + NBUF < units_per_w:
            zero(accs[p])
            fire(u + NBUF, accs[p], sems[p])

    pltpu.sync_copy(out_v, out_hbm.at[pl.ds(bag0, bags_per_w)])


@jax.jit
def _bag_of_words(ids3, table):
    batch = ids3.shape[1] * ids3.shape[2]
    bags_per_w = batch // NUM_WORKERS
    units_per_w = bags_per_w // UNIT_BAGS
    grid_kernel = functools.partial(
        pl.kernel,
        mesh=plsc.VectorSubcoreMesh(core_axis_name="c", subcore_axis_name="s"),
        out_type=jax.ShapeDtypeStruct((batch, D), jnp.float32),
        scratch_types=[
            pltpu.VMEM((SEQ, units_per_w, UNIT_BAGS), jnp.int32),
            pltpu.VMEM((UNIT_BAGS, D), jnp.float32),
            pltpu.VMEM((UNIT_BAGS, D), jnp.float32),
            pltpu.VMEM((bags_per_w, D), jnp.float32),
            pltpu.SemaphoreType.DMA,
            pltpu.SemaphoreType.DMA,
        ],
        compiler_params=pltpu.CompilerParams(use_tc_tiling_on_sc=False),
    )
    return grid_kernel(functools.partial(_bag_kernel, batch))(ids3, table)


def kernel(token_ids, table):
    batch, seq = token_ids.shape
    assert seq == SEQ and table.shape[1] == D
    assert batch % (NUM_WORKERS * UNIT_BAGS) == 0
    ids3 = token_ids.astype(jnp.int32).T.reshape(
        SEQ, batch // UNIT_BAGS, UNIT_BAGS)
    return _bag_of_words(ids3, table)
